# baseline (speedup 1.0000x reference)
import jax
import jax.numpy as jnp
from jax import lax
from jax.experimental import pallas as pl
from jax.experimental.pallas import tpu as pltpu
from jax.experimental.pallas import tpu_sc as plsc

NQ = 8
D = 64
AD = 128
K = 16
NROWS = 1000000
NB = 4096
NBLK = (NROWS + NB - 1) // NB
NPAD = NBLK * NB
FB = 512
NFINE = NPAD // FB
BMPAD = 2048
BIG = 1 << 30


def _fine_mins(x):
    return jnp.concatenate(
        [jnp.min(x[:, f * FB:(f + 1) * FB], axis=1, keepdims=True)
         for f in range(NB // FB)], axis=1)


def _dist_body(q2row_ref, obs_ref, db_ref, d_ref, bm_ref):
    i = pl.program_id(0)
    blk = db_ref[...]
    m1t = lax.dot_general(blk, obs_ref[...], (((1,), (1,)), ((), ())),
                          preferred_element_type=jnp.float32)
    d2 = jnp.sum(blk * blk, axis=1, keepdims=True)
    sqt = jnp.maximum(q2row_ref[...] - 2.0 * m1t + d2, 0.0)
    dist = jnp.transpose(jnp.sqrt(sqt), (1, 0))
    d_ref[...] = dist
    bm_ref[...] = _fine_mins(dist).reshape(1, NQ, NB // FB)

    @pl.when(i == NBLK - 1)
    def _tail():
        col = lax.broadcasted_iota(jnp.int32, (NQ, NB), 1) + i * NB
        dm = jnp.where(col < NROWS, dist, jnp.inf)
        d_ref[...] = dm
        bm_ref[...] = _fine_mins(dm).reshape(1, NQ, NB // FB)


def _fine_mins(x):
    return jnp.concatenate(
        [jnp.min(x[:, f * FB:(f + 1) * FB], axis=1, keepdims=True)
         for f in range(NB // FB)], axis=1)


def _distances(q2row, observations, database):
    return pl.pallas_call(
        _dist_body,
        grid=(NBLK,),
        in_specs=[
            pl.BlockSpec((1, NQ), lambda i: (0, 0)),
            pl.BlockSpec((NQ, D), lambda i: (0, 0)),
            pl.BlockSpec((NB, D), lambda i: (i, 0)),
        ],
        out_specs=[
            pl.BlockSpec((NQ, NB), lambda i: (0, i)),
            pl.BlockSpec((1, NQ, NB // FB), lambda i: (i, 0, 0)),
        ],
        out_shape=[
            jax.ShapeDtypeStruct((NQ, NPAD), jnp.float32),
            jax.ShapeDtypeStruct((NBLK, NQ, NB // FB), jnp.float32),
        ],
        compiler_params=pltpu.CompilerParams(
            dimension_semantics=("arbitrary",)),
    )(q2row, observations, database)


def _lanes16():
    return lax.iota(jnp.int32, 16)


def _take16(v, idx):
    return lax.gather(
        v, idx.reshape(16, 1),
        lax.GatherDimensionNumbers(offset_dims=(), collapsed_slice_dims=(0,),
                                   start_index_map=(0,)),
        (1,), mode=lax.GatherScatterMode.PROMISE_IN_BOUNDS)


def _mintree_v(v):
    lanes = _lanes16()
    for sh in (8, 4, 2, 1):
        v = jnp.minimum(v, _take16(v, lanes ^ sh))
    return v


def _sortnet16(v, c):
    lanes = _lanes16()
    for kk in (2, 4, 8, 16):
        j = kk >> 1
        while j:
            perm = lanes ^ j
            pv = _take16(v, perm)
            pc = _take16(c, perm)
            lt = (pv < v) | ((pv == v) & (pc < c))
            ge = (pv > v) | ((pv == v) & (pc > c))
            a = (lanes & kk) == 0
            an = (lanes & kk) != 0
            bb = (lanes & j) == 0
            bn = (lanes & j) != 0
            up = (a & bb) | (an & bn)
            dn = (a & bn) | (an & bb)
            tp = (up & lt) | (dn & ge)
            v = jnp.where(tp, pv, v)
            c = jnp.where(tp, pc, c)
            j >>= 1
    return v, c


def _bitonic_merge16(v, c):
    lanes = _lanes16()
    for j in (8, 4, 2, 1):
        perm = lanes ^ j
        pv = _take16(v, perm)
        pc = _take16(c, perm)
        lt = (pv < v) | ((pv == v) & (pc < c))
        ge = (pv > v) | ((pv == v) & (pc > c))
        lower = (lanes & j) == 0
        upper = (lanes & j) != 0
        tp = (lower & lt) | (upper & ge)
        v = jnp.where(tp, pv, v)
        c = jnp.where(tp, pc, c)
    return v, c


def _scan_chunks(nchunks, load_chunk, val_ref, code_ref):
    def step(k, carry):
        v, c = load_chunk(k)
        mv = _mintree_v(v)
        rv0 = val_ref[...]
        t = rv0[15]

        @pl.when(mv[0] < t)
        def _merge():
            sv, sc = _sortnet16(v, c)
            rv = val_ref[...]
            rc = code_ref[...]
            svr = lax.rev(sv, (0,))
            scr = lax.rev(sc, (0,))
            lt = (svr < rv) | ((svr == rv) & (scr < rc))
            lov = jnp.where(lt, svr, rv)
            loc = jnp.where(lt, scr, rc)
            nv, nc = _bitonic_merge16(lov, loc)
            val_ref[...] = nv
            code_ref[...] = nc

        return carry

    lax.fori_loop(0, nchunks, step, 0)


def _select_body(bm_hbm, d_hbm, labels_hbm, sq_out, act_out,
                 bm_v, rowid_v, bid_v, rows_v, idx_v, val16_v,
                 code16_v, act_v, sem):
    wid = lax.axis_index("s") * 2 + lax.axis_index("c")
    q = wid

    @pl.when(q < NQ)
    def _():
        inf = jnp.float32(jnp.inf)
        lanes = lax.iota(jnp.int32, 16)
        pltpu.sync_copy(bm_hbm.at[q], bm_v)

        val16_v[...] = jnp.full((16,), inf, jnp.float32)
        code16_v[...] = jnp.full((16,), BIG, jnp.int32)

        def load_bm(k):
            return bm_v[pl.ds(k * 16, 16)], k * 16 + lanes

        _scan_chunks(BMPAD // 16, load_bm, val16_v, code16_v)

        b = code16_v[...]
        for kk in (2, 4, 8, 16):
            j = kk >> 1
            while j:
                p = _take16(b, lanes ^ j)
                up = ((lanes & kk) == 0) ^ ((lanes & j) != 0)
                b = jnp.where(up, jnp.minimum(b, p), jnp.maximum(b, p))
                j >>= 1
        bid_v[...] = b
        rowid_v[...] = q * NFINE + b

        pltpu.async_copy(d_hbm.at[rowid_v], rows_v, sem).wait()

        val16_v[...] = jnp.full((16,), inf, jnp.float32)
        code16_v[...] = jnp.full((16,), BIG, jnp.int32)

        def load_row(m):
            v = rows_v[m >> 5, pl.ds((m & 31) * 16, 16)]
            return v, m * 16 + lanes

        _scan_chunks(K * (FB // 16), load_row, val16_v, code16_v)

        rc = code16_v[...]
        bsel = _take16(bid_v[...], rc >> 9)
        g = bsel * FB + (rc & (FB - 1))

        idx_v[...] = g
        pltpu.async_copy(labels_hbm.at[idx_v], act_v, sem).wait()

        pltpu.sync_copy(val16_v, sq_out.at[q])
        pltpu.sync_copy(act_v, act_out.at[pl.ds(q * K, K)])


def _select(bm, d2d, labels):
    mesh = plsc.VectorSubcoreMesh(core_axis_name="c", subcore_axis_name="s")
    fn = pl.kernel(
        _select_body,
        out_type=[
            jax.ShapeDtypeStruct((NQ, K), jnp.float32),
            jax.ShapeDtypeStruct((NQ * K, AD), jnp.float32),
        ],
        mesh=mesh,
        scratch_types=[
            pltpu.VMEM((BMPAD,), jnp.float32),
            pltpu.VMEM((K,), jnp.int32),
            pltpu.VMEM((K,), jnp.int32),
            pltpu.VMEM((K, FB), jnp.float32),
            pltpu.VMEM((K,), jnp.int32),
            pltpu.VMEM((K,), jnp.float32),
            pltpu.VMEM((K,), jnp.int32),
            pltpu.VMEM((K, AD), jnp.float32),
            pltpu.SemaphoreType.DMA,
        ],
    )
    return fn(bm, d2d, labels)


def _final_body(sq_ref, act_ref, out_ref):
    p = jnp.exp(-sq_ref[...])
    total = jnp.sum(p)
    pb = jnp.concatenate([p] * NQ, axis=1)
    col = lax.broadcasted_iota(jnp.int32, (NQ, NQ * K), 1)
    row = lax.broadcasted_iota(jnp.int32, (NQ, NQ * K), 0)
    w = jnp.where((col >> 4) == row, pb, 0.0) / total
    out_ref[...] = lax.dot_general(w, act_ref[...], (((1,), (0,)), ((), ())),
                                   preferred_element_type=jnp.float32)


def _finalize(sqsel, acts):
    return pl.pallas_call(
        _final_body,
        out_shape=jax.ShapeDtypeStruct((NQ, AD), jnp.float32),
    )(sqsel, acts)


def kernel(observations, database, database_labels, topk):
    q2row = jnp.sum(observations * observations, axis=1,
                    keepdims=True).reshape(1, NQ)
    d, bm3 = _distances(q2row, observations, database)
    d2d = d.reshape(NQ * NFINE, FB)
    bm = jnp.transpose(bm3, (1, 0, 2)).reshape(NQ, NFINE)
    bm = jnp.pad(bm, ((0, 0), (0, BMPAD - NFINE)),
                 constant_values=jnp.inf)
    sqsel, acts = _select(bm, d2d, database_labels)
    out = _finalize(sqsel, acts)
    return out.reshape(-1, 8, 16)


# speedup vs baseline: 1.1074x; 1.1074x over previous
import jax
import jax.numpy as jnp
from jax import lax
from jax.experimental import pallas as pl
from jax.experimental.pallas import tpu as pltpu
from jax.experimental.pallas import tpu_sc as plsc

NQ = 8
D = 64
AD = 128
K = 16
NROWS = 1000000
NB = 16384
NBLK = (NROWS + NB - 1) // NB
NPAD = NBLK * NB
FB = 512
NFINE = NPAD // FB
BMPAD = 2048
BIG = 1 << 30


def _fine_mins(x):
    return jnp.concatenate(
        [jnp.min(x[:, f * FB:(f + 1) * FB], axis=1, keepdims=True)
         for f in range(NB // FB)], axis=1)


def _dist_body(q2row_ref, obs_ref, db_ref, d_ref, bm_ref):
    i = pl.program_id(0)
    blk = db_ref[...]
    m1t = lax.dot_general(blk, obs_ref[...], (((1,), (1,)), ((), ())),
                          preferred_element_type=jnp.float32)
    d2 = jnp.sum(blk * blk, axis=1, keepdims=True)
    sqt = jnp.maximum(q2row_ref[...] - 2.0 * m1t + d2, 0.0)
    dist = jnp.transpose(jnp.sqrt(sqt), (1, 0))
    d_ref[...] = dist
    bm_ref[...] = _fine_mins(dist).reshape(1, NQ, NB // FB)

    @pl.when(i == NBLK - 1)
    def _tail():
        col = lax.broadcasted_iota(jnp.int32, (NQ, NB), 1) + i * NB
        dm = jnp.where(col < NROWS, dist, jnp.inf)
        d_ref[...] = dm
        bm_ref[...] = _fine_mins(dm).reshape(1, NQ, NB // FB)


def _fine_mins(x):
    return jnp.concatenate(
        [jnp.min(x[:, f * FB:(f + 1) * FB], axis=1, keepdims=True)
         for f in range(NB // FB)], axis=1)


def _distances(q2row, observations, database):
    return pl.pallas_call(
        _dist_body,
        grid=(NBLK,),
        in_specs=[
            pl.BlockSpec((1, NQ), lambda i: (0, 0)),
            pl.BlockSpec((NQ, D), lambda i: (0, 0)),
            pl.BlockSpec((NB, D), lambda i: (i, 0)),
        ],
        out_specs=[
            pl.BlockSpec((NQ, NB), lambda i: (0, i)),
            pl.BlockSpec((1, NQ, NB // FB), lambda i: (i, 0, 0)),
        ],
        out_shape=[
            jax.ShapeDtypeStruct((NQ, NPAD), jnp.float32),
            jax.ShapeDtypeStruct((NBLK, NQ, NB // FB), jnp.float32),
        ],
        compiler_params=pltpu.CompilerParams(
            dimension_semantics=("arbitrary",)),
    )(q2row, observations, database)


def _lanes16():
    return lax.iota(jnp.int32, 16)


def _take16(v, idx):
    return lax.gather(
        v, idx.reshape(16, 1),
        lax.GatherDimensionNumbers(offset_dims=(), collapsed_slice_dims=(0,),
                                   start_index_map=(0,)),
        (1,), mode=lax.GatherScatterMode.PROMISE_IN_BOUNDS)


def _mintree_v(v):
    lanes = _lanes16()
    for sh in (8, 4, 2, 1):
        v = jnp.minimum(v, _take16(v, lanes ^ sh))
    return v


def _sortnet16(v, c):
    lanes = _lanes16()
    for kk in (2, 4, 8, 16):
        j = kk >> 1
        while j:
            perm = lanes ^ j
            pv = _take16(v, perm)
            pc = _take16(c, perm)
            lt = (pv < v) | ((pv == v) & (pc < c))
            ge = (pv > v) | ((pv == v) & (pc > c))
            a = (lanes & kk) == 0
            an = (lanes & kk) != 0
            bb = (lanes & j) == 0
            bn = (lanes & j) != 0
            up = (a & bb) | (an & bn)
            dn = (a & bn) | (an & bb)
            tp = (up & lt) | (dn & ge)
            v = jnp.where(tp, pv, v)
            c = jnp.where(tp, pc, c)
            j >>= 1
    return v, c


def _bitonic_merge16(v, c):
    lanes = _lanes16()
    for j in (8, 4, 2, 1):
        perm = lanes ^ j
        pv = _take16(v, perm)
        pc = _take16(c, perm)
        lt = (pv < v) | ((pv == v) & (pc < c))
        ge = (pv > v) | ((pv == v) & (pc > c))
        lower = (lanes & j) == 0
        upper = (lanes & j) != 0
        tp = (lower & lt) | (upper & ge)
        v = jnp.where(tp, pv, v)
        c = jnp.where(tp, pc, c)
    return v, c


def _scan_chunks(nchunks, load_chunk, val_ref, code_ref):
    def step(k, carry):
        v, c = load_chunk(k)
        mv = _mintree_v(v)
        rv0 = val_ref[...]
        t = rv0[15]

        @pl.when(mv[0] < t)
        def _merge():
            sv, sc = _sortnet16(v, c)
            rv = val_ref[...]
            rc = code_ref[...]
            svr = lax.rev(sv, (0,))
            scr = lax.rev(sc, (0,))
            lt = (svr < rv) | ((svr == rv) & (scr < rc))
            lov = jnp.where(lt, svr, rv)
            loc = jnp.where(lt, scr, rc)
            nv, nc = _bitonic_merge16(lov, loc)
            val_ref[...] = nv
            code_ref[...] = nc

        return carry

    lax.fori_loop(0, nchunks, step, 0)


def _select_body(bm_hbm, d_hbm, labels_hbm, sq_out, act_out,
                 bm_v, rowid_v, bid_v, rows_v, idx_v, val16_v,
                 code16_v, act_v, sem):
    wid = lax.axis_index("s") * 2 + lax.axis_index("c")
    q = wid

    @pl.when(q < NQ)
    def _():
        inf = jnp.float32(jnp.inf)
        lanes = lax.iota(jnp.int32, 16)
        pltpu.sync_copy(bm_hbm.at[q], bm_v)

        val16_v[...] = jnp.full((16,), inf, jnp.float32)
        code16_v[...] = jnp.full((16,), BIG, jnp.int32)

        def load_bm(k):
            return bm_v[pl.ds(k * 16, 16)], k * 16 + lanes

        _scan_chunks(BMPAD // 16, load_bm, val16_v, code16_v)

        b = code16_v[...]
        for kk in (2, 4, 8, 16):
            j = kk >> 1
            while j:
                p = _take16(b, lanes ^ j)
                up = ((lanes & kk) == 0) ^ ((lanes & j) != 0)
                b = jnp.where(up, jnp.minimum(b, p), jnp.maximum(b, p))
                j >>= 1
        bid_v[...] = b
        rowid_v[...] = q * NFINE + b

        pltpu.async_copy(d_hbm.at[rowid_v], rows_v, sem).wait()

        val16_v[...] = jnp.full((16,), inf, jnp.float32)
        code16_v[...] = jnp.full((16,), BIG, jnp.int32)

        def load_row(m):
            v = rows_v[m >> 5, pl.ds((m & 31) * 16, 16)]
            return v, m * 16 + lanes

        _scan_chunks(K * (FB // 16), load_row, val16_v, code16_v)

        rc = code16_v[...]
        bsel = _take16(bid_v[...], rc >> 9)
        g = bsel * FB + (rc & (FB - 1))

        idx_v[...] = g
        pltpu.async_copy(labels_hbm.at[idx_v], act_v, sem).wait()

        pltpu.sync_copy(val16_v, sq_out.at[q])
        pltpu.sync_copy(act_v, act_out.at[pl.ds(q * K, K)])


def _select(bm, d2d, labels):
    mesh = plsc.VectorSubcoreMesh(core_axis_name="c", subcore_axis_name="s")
    fn = pl.kernel(
        _select_body,
        out_type=[
            jax.ShapeDtypeStruct((NQ, K), jnp.float32),
            jax.ShapeDtypeStruct((NQ * K, AD), jnp.float32),
        ],
        mesh=mesh,
        scratch_types=[
            pltpu.VMEM((BMPAD,), jnp.float32),
            pltpu.VMEM((K,), jnp.int32),
            pltpu.VMEM((K,), jnp.int32),
            pltpu.VMEM((K, FB), jnp.float32),
            pltpu.VMEM((K,), jnp.int32),
            pltpu.VMEM((K,), jnp.float32),
            pltpu.VMEM((K,), jnp.int32),
            pltpu.VMEM((K, AD), jnp.float32),
            pltpu.SemaphoreType.DMA,
        ],
    )
    return fn(bm, d2d, labels)


def _final_body(sq_ref, act_ref, out_ref):
    p = jnp.exp(-sq_ref[...])
    total = jnp.sum(p)
    pb = jnp.concatenate([p] * NQ, axis=1)
    col = lax.broadcasted_iota(jnp.int32, (NQ, NQ * K), 1)
    row = lax.broadcasted_iota(jnp.int32, (NQ, NQ * K), 0)
    w = jnp.where((col >> 4) == row, pb, 0.0) / total
    out_ref[...] = lax.dot_general(w, act_ref[...], (((1,), (0,)), ((), ())),
                                   preferred_element_type=jnp.float32)


def _finalize(sqsel, acts):
    return pl.pallas_call(
        _final_body,
        out_shape=jax.ShapeDtypeStruct((NQ, AD), jnp.float32),
    )(sqsel, acts)


def kernel(observations, database, database_labels, topk):
    q2row = jnp.sum(observations * observations, axis=1,
                    keepdims=True).reshape(1, NQ)
    d, bm3 = _distances(q2row, observations, database)
    d2d = d.reshape(NQ * NFINE, FB)
    bm = jnp.transpose(bm3, (1, 0, 2)).reshape(NQ, NFINE)
    bm = jnp.pad(bm, ((0, 0), (0, BMPAD - NFINE)),
                 constant_values=jnp.inf)
    sqsel, acts = _select(bm, d2d, database_labels)
    out = _finalize(sqsel, acts)
    return out.reshape(-1, 8, 16)
